# baseline (speedup 1.0000x reference)
import functools

import jax
import jax.numpy as jnp
from jax.experimental import pallas as pl
from jax.experimental.pallas import tpu as pltpu

_CODE_NUM = 8
_D = 3
_IDX = 1


def _interp_kernel(coords_ref, code_ref, out_ref, *, cn, c, tp):
    tq = tp // 4
    scaled = (coords_ref[...] + 1.0) * ((cn - 1) / 2.0)
    grid_i = jax.lax.broadcasted_iota(jnp.int32, (cn, tq), 0).astype(jnp.float32)

    accs = []
    for s in range(4):
        sc = scaled[s * 8:(s + 1) * 8, :]

        def hat(j):
            return jnp.maximum(0.0, 1.0 - jnp.abs(grid_i - sc[j:j + 1, :]))

        h0 = hat(0)
        h1 = hat(1)
        h2 = hat(2)

        w_low = (h1[:, None, :] * h0[None, :, :]).reshape(cn * cn, tq)

        a = jnp.dot(code_ref[...], w_low,
                    preferred_element_type=jnp.float32)

        acc = a[0:c, :] * h2[0:1, :]
        for i in range(1, cn):
            acc = acc + a[i * c:(i + 1) * c, :] * h2[i:i + 1, :]
        accs.append(acc)

    packed = jnp.concatenate(accs, axis=0)
    out_ref[...] = packed.T


@jax.jit
def kernel(coords, shape_code):
    b, p, d = coords.shape
    c = shape_code.shape[0]
    cn = _CODE_NUM
    nblk = cn ** d

    npts = b * p
    tp = 8192
    npts_pad = pl.cdiv(npts, tp) * tp

    cf = jnp.pad(coords.reshape(npts, d).astype(jnp.float32),
                 ((0, npts_pad - npts), (0, 8 - d)))
    coords_t = (cf.reshape(npts_pad // 4, 4, 8)
                .transpose(1, 2, 0)
                .reshape(32, npts_pad // 4))

    code = jax.lax.slice_in_dim(shape_code, _IDX * nblk, (_IDX + 1) * nblk, axis=1)
    code_r = (code.astype(jnp.float32)
              .reshape(c, cn, cn * cn)
              .transpose(1, 0, 2)
              .reshape(cn * c, cn * cn))

    kernel_fn = functools.partial(_interp_kernel, cn=cn, c=c, tp=tp)

    out = pl.pallas_call(
        kernel_fn,
        out_shape=jax.ShapeDtypeStruct((npts_pad // 4, 4 * c), jnp.float32),
        grid=(npts_pad // tp,),
        in_specs=[
            pl.BlockSpec((32, tp // 4), lambda i: (0, i)),
            pl.BlockSpec((cn * c, cn * cn), lambda i: (0, 0)),
        ],
        out_specs=pl.BlockSpec((tp // 4, 4 * c), lambda i: (i, 0)),
        compiler_params=pltpu.CompilerParams(
            dimension_semantics=("parallel",),
            vmem_limit_bytes=64 * 1024 * 1024,
        ),
    )(coords_t, code_r)

    return out.reshape(npts_pad, c)[:npts].reshape(b, p, c)


# speedup vs baseline: 7.0486x; 7.0486x over previous
import functools

import jax
import jax.numpy as jnp
from jax.experimental import pallas as pl
from jax.experimental.pallas import tpu as pltpu

_CODE_NUM = 8
_D = 3
_IDX = 1


def _interp_kernel(coords_ref, code_ref, out_ref, *, cn, c, tp):
    scaled = (coords_ref[...] + 1.0) * ((cn - 1) / 2.0)
    grid_i = jax.lax.broadcasted_iota(jnp.int32, (cn, tp), 0).astype(jnp.float32)

    def hat(j):
        return jnp.maximum(0.0, 1.0 - jnp.abs(grid_i - scaled[j:j + 1, :]))

    h0 = hat(0)
    h1 = hat(1)
    h2 = hat(2)

    w_low = (h1[:, None, :] * h0[None, :, :]).reshape(cn * cn, tp)

    a = jnp.dot(code_ref[...], w_low,
                preferred_element_type=jnp.float32)

    acc = a[0:c, :] * h2[0:1, :]
    for i in range(1, cn):
        acc = acc + a[i * c:(i + 1) * c, :] * h2[i:i + 1, :]

    out_ref[0, :, :] = acc.T


@jax.jit
def kernel(coords, shape_code):
    b, p, d = coords.shape
    c = shape_code.shape[0]
    cn = _CODE_NUM
    nblk = cn ** d

    npts = b * p
    tp = 8192
    if p % tp != 0:
        tp = 1024 if p % 1024 == 0 else p
    tiles_per_batch = p // tp

    coords_t = jnp.pad(coords.reshape(npts, d).T.astype(jnp.float32),
                       ((0, 8 - d), (0, 0)))

    code = jax.lax.slice_in_dim(shape_code, _IDX * nblk, (_IDX + 1) * nblk, axis=1)
    code_r = (code.astype(jnp.float32)
              .reshape(c, cn, cn * cn)
              .transpose(1, 0, 2)
              .reshape(cn * c, cn * cn))

    kernel_fn = functools.partial(_interp_kernel, cn=cn, c=c, tp=tp)

    out = pl.pallas_call(
        kernel_fn,
        out_shape=jax.ShapeDtypeStruct((b, p, c), jnp.float32),
        grid=(npts // tp,),
        in_specs=[
            pl.BlockSpec((8, tp), lambda i: (0, i)),
            pl.BlockSpec((cn * c, cn * cn), lambda i: (0, 0)),
        ],
        out_specs=pl.BlockSpec(
            (1, tp, c),
            lambda i, _t=tiles_per_batch: (i // _t, i % _t, 0)),
        compiler_params=pltpu.CompilerParams(
            dimension_semantics=("parallel",),
            vmem_limit_bytes=64 * 1024 * 1024,
        ),
    )(coords_t, code_r)

    return out


# speedup vs baseline: 10.6933x; 1.5171x over previous
import functools

import jax
import jax.numpy as jnp
from jax.experimental import pallas as pl
from jax.experimental.pallas import tpu as pltpu

_CODE_NUM = 8
_D = 3
_IDX = 1


def _interp_kernel(coords_ref, code_ref, out_ref, *, cn, c, tp):
    scaled = (coords_ref[...] + 1.0) * ((cn - 1) / 2.0)
    grid_i = jax.lax.broadcasted_iota(jnp.int32, (cn, tp), 0).astype(jnp.float32)

    def hat(j):
        return jnp.maximum(0.0, 1.0 - jnp.abs(grid_i - scaled[j:j + 1, :]))

    h0 = hat(0)
    h1 = hat(1)
    h2 = hat(2)

    w_low = (h1[:, None, :] * h0[None, :, :]).reshape(cn * cn, tp)

    a = jnp.dot(code_ref[...], w_low,
                preferred_element_type=jnp.float32)

    acc = a[0:c, :] * h2[0:1, :]
    for i in range(1, cn):
        acc = acc + a[i * c:(i + 1) * c, :] * h2[i:i + 1, :]

    out_ref[...] = acc


@jax.jit
def kernel(coords, shape_code):
    b, p, d = coords.shape
    c = shape_code.shape[0]
    cn = _CODE_NUM
    nblk = cn ** d

    code = jax.lax.slice_in_dim(shape_code, _IDX * nblk, (_IDX + 1) * nblk, axis=1)
    code_r = (code.astype(jnp.float32)
              .reshape(c, cn, cn * cn)
              .transpose(1, 0, 2)
              .reshape(cn * c, cn * cn))

    n_chunks = 4
    if b % n_chunks != 0:
        n_chunks = 1
    bc = b // n_chunks
    npts = bc * p
    tp = 8192
    if npts % tp != 0:
        tp = 1024 if npts % 1024 == 0 else p
    kernel_fn = functools.partial(_interp_kernel, cn=cn, c=c, tp=tp)

    outs = []
    for k in range(n_chunks):
        ck = jax.lax.slice_in_dim(coords, k * bc, (k + 1) * bc, axis=0)
        coords_t = jnp.pad(ck.reshape(npts, d).T.astype(jnp.float32),
                           ((0, 8 - d), (0, 0)))

        out = pl.pallas_call(
            kernel_fn,
            out_shape=jax.ShapeDtypeStruct((c, npts), jnp.float32),
            grid=(npts // tp,),
            in_specs=[
                pl.BlockSpec((8, tp), lambda i: (0, i)),
                pl.BlockSpec((cn * c, cn * cn), lambda i: (0, 0)),
            ],
            out_specs=pl.BlockSpec((c, tp), lambda i: (0, i)),
            compiler_params=pltpu.CompilerParams(
                dimension_semantics=("parallel",),
                vmem_limit_bytes=64 * 1024 * 1024,
            ),
        )(coords_t, code_r)

        outs.append(out.T.reshape(bc, p, c))

    return jnp.concatenate(outs, axis=0) if n_chunks > 1 else outs[0]


# speedup vs baseline: 18.6582x; 1.7449x over previous
import functools

import jax
import jax.numpy as jnp
from jax.experimental import pallas as pl
from jax.experimental.pallas import tpu as pltpu

_CODE_NUM = 8
_D = 3
_IDX = 1


def _interp_kernel(coords_ref, code_ref, out_ref, *, cn, c, tp):
    scaled = (coords_ref[...] + 1.0) * ((cn - 1) / 2.0)
    grid_i = jax.lax.broadcasted_iota(jnp.int32, (cn, tp), 0).astype(jnp.float32)

    def hat(j):
        return jnp.maximum(0.0, 1.0 - jnp.abs(grid_i - scaled[j:j + 1, :]))

    h0 = hat(0)
    h1 = hat(1)
    h2 = hat(2)

    w_low = (h1[:, None, :] * h0[None, :, :]).reshape(cn * cn, tp)

    a = jnp.dot(code_ref[...], w_low,
                preferred_element_type=jnp.float32)

    acc = a[0:c, :] * h2[0:1, :]
    for i in range(1, cn):
        acc = acc + a[i * c:(i + 1) * c, :] * h2[i:i + 1, :]

    out_ref[0, :, :] = acc


@jax.jit
def kernel(coords, shape_code):
    b, p, d = coords.shape
    c = shape_code.shape[0]
    cn = _CODE_NUM
    nblk = cn ** d

    code = jax.lax.slice_in_dim(shape_code, _IDX * nblk, (_IDX + 1) * nblk, axis=1)
    code_r = (code.astype(jnp.float32)
              .reshape(c, cn, cn * cn)
              .transpose(1, 0, 2)
              .reshape(cn * c, cn * cn))

    npts = b * p
    tp = 8192
    if p % tp != 0:
        tp = 1024 if p % 1024 == 0 else p
    tiles_per_batch = p // tp
    kernel_fn = functools.partial(_interp_kernel, cn=cn, c=c, tp=tp)

    coords_t = jnp.pad(coords.reshape(npts, d).T.astype(jnp.float32),
                       ((0, 8 - d), (0, 0)))

    out = pl.pallas_call(
        kernel_fn,
        out_shape=jax.ShapeDtypeStruct((b, c, p), jnp.float32),
        grid=(b * tiles_per_batch,),
        in_specs=[
            pl.BlockSpec((8, tp), lambda i: (0, i)),
            pl.BlockSpec((cn * c, cn * cn), lambda i: (0, 0)),
        ],
        out_specs=pl.BlockSpec(
            (1, c, tp),
            lambda i, _t=tiles_per_batch: (i // _t, 0, i % _t)),
        compiler_params=pltpu.CompilerParams(
            dimension_semantics=("parallel",),
            vmem_limit_bytes=64 * 1024 * 1024,
        ),
    )(coords_t, code_r)

    return out.transpose(0, 2, 1)


# speedup vs baseline: 20.0299x; 1.0735x over previous
import functools

import jax
import jax.numpy as jnp
from jax.experimental import pallas as pl
from jax.experimental.pallas import tpu as pltpu

_CODE_NUM = 8
_D = 3
_IDX = 1


def _interp_kernel(coords_ref, code_ref, out_ref, *, cn, c, tp):
    scaled = (coords_ref[...] + 1.0) * ((cn - 1) / 2.0)
    grid_i = jax.lax.broadcasted_iota(jnp.int32, (cn, tp), 0).astype(jnp.float32)

    def hat(j):
        return jnp.maximum(0.0, 1.0 - jnp.abs(grid_i - scaled[j:j + 1, :]))

    h0 = hat(0)
    h1 = hat(1)
    h2 = hat(2)

    w_low = (h1[:, None, :] * h0[None, :, :]).reshape(cn * cn, tp)

    a = jnp.dot(code_ref[...], w_low,
                preferred_element_type=jnp.float32)

    acc = a[0:c, :] * h2[0:1, :]
    for i in range(1, cn):
        acc = acc + a[i * c:(i + 1) * c, :] * h2[i:i + 1, :]

    out_ref[0, :, :] = acc


@jax.jit
def kernel(coords, shape_code):
    b, p, d = coords.shape
    c = shape_code.shape[0]
    cn = _CODE_NUM
    nblk = cn ** d

    code = jax.lax.slice_in_dim(shape_code, _IDX * nblk, (_IDX + 1) * nblk, axis=1)
    code_r = (code.astype(jnp.float32)
              .reshape(c, cn, cn * cn)
              .transpose(1, 0, 2)
              .reshape(cn * c, cn * cn))

    npts = b * p
    tp = 16384
    if p % tp != 0:
        tp = 1024 if p % 1024 == 0 else p
    tiles_per_batch = p // tp
    kernel_fn = functools.partial(_interp_kernel, cn=cn, c=c, tp=tp)

    coords_t = jnp.pad(coords.reshape(npts, d).T.astype(jnp.float32),
                       ((0, 8 - d), (0, 0)))

    out = pl.pallas_call(
        kernel_fn,
        out_shape=jax.ShapeDtypeStruct((b, c, p), jnp.float32),
        grid=(b * tiles_per_batch,),
        in_specs=[
            pl.BlockSpec((8, tp), lambda i: (0, i)),
            pl.BlockSpec((cn * c, cn * cn), lambda i: (0, 0)),
        ],
        out_specs=pl.BlockSpec(
            (1, c, tp),
            lambda i, _t=tiles_per_batch: (i // _t, 0, i % _t)),
        compiler_params=pltpu.CompilerParams(
            dimension_semantics=("parallel",),
            vmem_limit_bytes=64 * 1024 * 1024,
        ),
    )(coords_t, code_r)

    return out.transpose(0, 2, 1)
